# baseline (speedup 1.0000x reference)
import functools

import jax
import jax.numpy as jnp
from jax import lax
from jax.experimental import pallas as pl
from jax.experimental.pallas import tpu as pltpu
from jax.experimental.pallas import tpu_sc as plsc

NC = 2
NS = 16
BN = 1000
KE = 1280
CK = 80
CKD = 40


def _stage_a_body(x_ref, pos_ref, W_in_ref, b_in_ref, W_src_ref, W_dst_ref,
                  W_lin_ref, Wp1_ref, bp1_ref, Wp2_ref, bp2_ref,
                  Wa1_ref, ba1_ref, Wa2_ref, ba2_ref,
                  t_ref, sL_ref, alphaL_ref, cmax_ref):
    pid = pl.program_id(0)
    D = x_ref.shape[1]
    h = jnp.maximum(x_ref[...] @ W_in_ref[...] + b_in_ref[...], 0.0)
    t_ref[:, 0:D] = h
    t_ref[:, D:2 * D] = pos_ref[...] @ Wp1_ref[...]
    d0 = jnp.maximum(jnp.maximum(bp1_ref[...], 0.0) @ Wp2_ref[...]
                     + bp2_ref[...], 0.0)
    sL_ref[...] = h @ W_lin_ref[...] + d0
    t = h @ W_dst_ref[...] - h @ W_src_ref[...] + d0
    a1 = jnp.maximum(t @ Wa1_ref[...] + ba1_ref[...], 0.0)
    aL = jnp.maximum(a1 @ Wa2_ref[...] + ba2_ref[...], 0.0)
    alphaL_ref[...] = aL

    @pl.when(pid == 0)
    def _():
        cmax_ref[...] = jnp.zeros_like(cmax_ref)

    cm = jnp.max(aL, axis=0, keepdims=True)
    cmax_ref[...] = jnp.maximum(cmax_ref[...], jnp.broadcast_to(cm, cmax_ref.shape))


def _stage_a(n, x, pos8, W_in, b_in, W_src, W_dst, W_lin, Wp1_p, bp1, Wp2, bp2,
             Wa1, ba1, Wa2, ba2):
    D = x.shape[1]
    H = bp1.shape[1]
    grid = (n // BN,)
    full = lambda shape: pl.BlockSpec(shape, lambda i: (0, 0))
    row = lambda w: pl.BlockSpec((BN, w), lambda i: (i, 0))
    return pl.pallas_call(
        _stage_a_body,
        grid=grid,
        in_specs=[row(D), pl.BlockSpec((BN, 8), lambda i: (i, 0)),
                  full((D, D)), full((1, D)), full((D, D)), full((D, D)),
                  full((D, D)), full((8, D)), full((1, H)), full((H, D)),
                  full((1, D)), full((D, H)), full((1, H)), full((H, D)),
                  full((1, D))],
        out_specs=[row(2 * D), row(D), row(D),
                   pl.BlockSpec((8, D), lambda i: (0, 0))],
        out_shape=[jax.ShapeDtypeStruct((n, 2 * D), jnp.float32),
                   jax.ShapeDtypeStruct((n, D), jnp.float32),
                   jax.ShapeDtypeStruct((n, D), jnp.float32),
                   jax.ShapeDtypeStruct((8, D), jnp.float32)],
    )(x, pos8, W_in, b_in, W_src, W_dst, W_lin, Wp1_p, bp1, Wp2, bp2,
      Wa1, ba1, Wa2, ba2)


def _gather_sc(tbl, ijt, E):
    n2, TW = tbl.shape
    D = TW // 2
    Hq = 64
    EW = E // (NC * NS)
    NCH = EW // CK

    @functools.partial(
        pl.kernel,
        out_type=[jax.ShapeDtypeStruct((E, D), jnp.float32),
                  jax.ShapeDtypeStruct((E, D), jnp.float32),
                  jax.ShapeDtypeStruct((E, Hq), jnp.float32)],
        mesh=plsc.VectorSubcoreMesh(core_axis_name="c", subcore_axis_name="s"),
        scratch_types=[pltpu.VMEM((2, CK), jnp.int32),
                       pltpu.VMEM((2, CK), jnp.int32),
                       pltpu.VMEM((2, CK), jnp.int32),
                       pltpu.VMEM((2, CK), jnp.int32),
                       pltpu.VMEM((CK, TW), jnp.float32),
                       pltpu.VMEM((CK, TW), jnp.float32),
                       pltpu.VMEM((CK, TW), jnp.float32),
                       pltpu.VMEM((CK, TW), jnp.float32),
                       pltpu.VMEM((CK, Hq), jnp.float32),
                       pltpu.VMEM((CK, Hq), jnp.float32),
                       pltpu.SemaphoreType.DMA,
                       pltpu.SemaphoreType.DMA,
                       pltpu.SemaphoreType.DMA,
                       pltpu.SemaphoreType.DMA,
                       pltpu.SemaphoreType.DMA,
                       pltpu.SemaphoreType.DMA],
    )
    def k(tbl_hbm, ijt_hbm, hi_hbm, hj_hbm, qd_hbm,
          ij0, ij1, ij2, ij3, bi0, bi1, bj0, bj1, qd0, qd1,
          gs0, gs1, ws0, ws1, js0, js1):
        ijv = (ij0, ij1, ij2, ij3)
        bi = (bi0, bi1)
        bj = (bj0, bj1)
        qdv = (qd0, qd1)
        gs = (gs0, gs1)
        ws = (ws0, ws1)
        js = (js0, js1)
        wid = lax.axis_index("s") * NC + lax.axis_index("c")
        base = wid * EW
        bch = wid * NCH

        def fire_ij(kk, sp, s4):
            pltpu.async_copy(ijt_hbm.at[bch + kk], ijv[s4], js[sp])

        def drain_ij(sp, s4):
            pltpu.make_async_copy(ijt_hbm.at[bch], ijv[s4], js[sp]).wait()

        def fire_gathers(kk, s2, s4):
            pltpu.async_copy(tbl_hbm.at[ijv[s4].at[0]], bi[s2], gs[s2])
            pltpu.async_copy(tbl_hbm.at[ijv[s4].at[1]], bj[s2], gs[s2])

        def drain_gathers(s2, s4):
            pltpu.make_async_copy(tbl_hbm.at[ijv[s4].at[0]], bi[s2],
                                  gs[s2]).wait()
            pltpu.make_async_copy(tbl_hbm.at[ijv[s4].at[1]], bj[s2],
                                  gs[s2]).wait()

        def qd_compute(s2):
            def rowfn(r, cc):
                for rr in (2 * r, 2 * r + 1):
                    for g in range(Hq // 16):
                        o = pl.ds(D + g * 16, 16)
                        qdv[s2][rr, pl.ds(g * 16, 16)] = \
                            bi[s2][rr, o] - bj[s2][rr, o]
                return cc

            lax.fori_loop(0, CK // 2, rowfn, 0)

        def fire_writes(kk, s2):
            e0 = base + kk * CK
            pltpu.async_copy(bi[s2].at[pl.ds(0, CK), pl.ds(0, D)],
                             hi_hbm.at[pl.ds(e0, CK)], ws[s2])
            pltpu.async_copy(bj[s2].at[pl.ds(0, CK), pl.ds(0, D)],
                             hj_hbm.at[pl.ds(e0, CK)], ws[s2])
            pltpu.async_copy(qdv[s2], qd_hbm.at[pl.ds(e0, CK)], ws[s2])

        def drain_writes(s2):
            pltpu.make_async_copy(bi[s2].at[pl.ds(0, CK), pl.ds(0, D)],
                                  hi_hbm.at[pl.ds(0, CK)], ws[s2]).wait()
            pltpu.make_async_copy(bj[s2].at[pl.ds(0, CK), pl.ds(0, D)],
                                  hj_hbm.at[pl.ds(0, CK)], ws[s2]).wait()
            pltpu.make_async_copy(qdv[s2], qd_hbm.at[pl.ds(0, CK)],
                                  ws[s2]).wait()

        def phase(kk, s2, s4, first):
            drain_gathers(s2, s4)
            qd_compute(s2)
            fire_writes(kk, s2)
            nx1 = kk + 1

            @pl.when(nx1 < NCH)
            def _():
                drain_ij(1 - s2, (s4 + 1) % 4)
                if not first:
                    drain_writes(1 - s2)
                fire_gathers(nx1, 1 - s2, (s4 + 1) % 4)

            nx2 = kk + 2

            @pl.when(nx2 < NCH)
            def _():
                fire_ij(nx2, s2, (s4 + 2) % 4)

        fire_ij(0, 0, 0)
        fire_ij(1, 1, 1)
        drain_ij(0, 0)
        fire_gathers(0, 0, 0)
        phase(0, 0, 0, True)
        phase(1, 1, 1, False)
        phase(2, 0, 2, False)
        phase(3, 1, 3, False)

        def body(g, cc):
            for j in range(4):
                phase(4 * g + j, j % 2, j, False)
            return cc

        lax.fori_loop(1, NCH // 4, body, 0)
        for kk in range(4 * (NCH // 4), NCH):
            phase(kk, kk % 2, kk % 4, False)
        drain_writes(0)
        drain_writes(1)

    return k(tbl, ijt)


def _stage_c_body(hi_ref, hj_ref, qd_ref, W_src_ref, W_dst_ref,
                  W_lin_ref, bp1_ref, Wp2_ref, bp2_ref, Wa1_ref, ba1_ref,
                  Wa2_ref, ba2_ref, comb_ref):
    hi = hi_ref[...]
    hj = hj_ref[...]
    g1 = qd_ref[...] + bp1_ref[...]
    delta = jnp.maximum(jnp.maximum(g1, 0.0) @ Wp2_ref[...] + bp2_ref[...], 0.0)
    t = hi @ W_dst_ref[...] - hj @ W_src_ref[...] + delta
    a1 = jnp.maximum(t @ Wa1_ref[...] + ba1_ref[...], 0.0)
    alpha = jnp.maximum(a1 @ Wa2_ref[...] + ba2_ref[...], 0.0)
    s = hj @ W_lin_ref[...] + delta
    Dh = alpha.shape[1] // 2
    comb_ref[0] = jnp.concatenate([alpha[:, 0:Dh], s[:, 0:Dh]], axis=1)
    comb_ref[1] = jnp.concatenate([alpha[:, Dh:], s[:, Dh:]], axis=1)


def _stage_c(hi, hj, qd, W_src, W_dst, W_lin, bp1, Wp2, bp2, Wa1, ba1,
             Wa2, ba2):
    E, D = hi.shape
    H = bp1.shape[1]
    grid = (E // KE,)
    full = lambda shape: pl.BlockSpec(shape, lambda i: (0, 0))
    row = lambda w: pl.BlockSpec((KE, w), lambda i: (i, 0))
    return pl.pallas_call(
        _stage_c_body,
        grid=grid,
        in_specs=[row(D), row(D), row(H),
                  full((D, D)), full((D, D)), full((D, D)), full((1, H)),
                  full((H, D)), full((1, D)), full((D, H)), full((1, H)),
                  full((H, D)), full((1, D))],
        out_specs=pl.BlockSpec((NC, KE, D), lambda i: (0, i, 0)),
        out_shape=jax.ShapeDtypeStruct((NC, E, D), jnp.float32),
    )(hi, hj, qd, W_src, W_dst, W_lin, bp1, Wp2, bp2, Wa1, ba1, Wa2, ba2)


def _scatter_sc(comb, ii2, cmax2, init):
    _, E, D = comb.shape
    Hh = D // 2
    n = init.shape[1]
    ESC = E // NS
    NCHD = ESC // CKD
    NPS = n // NS

    @functools.partial(
        pl.kernel,
        out_type=jax.ShapeDtypeStruct((NC, n, D), jnp.float32),
        mesh=plsc.VectorSubcoreMesh(core_axis_name="c", subcore_axis_name="s"),
        scratch_types=[pltpu.VMEM((1, CKD), jnp.int32),
                       pltpu.VMEM((1, CKD), jnp.int32),
                       pltpu.VMEM((1, CKD), jnp.int32),
                       pltpu.VMEM((1, CKD), jnp.int32),
                       pltpu.VMEM((CKD, D), jnp.float32),
                       pltpu.VMEM((CKD, D), jnp.float32),
                       pltpu.VMEM((CKD, D), jnp.float32),
                       pltpu.VMEM((CKD, D), jnp.float32),
                       pltpu.VMEM((Hh,), jnp.float32),
                       pltpu.VMEM_SHARED((n, D), jnp.float32),
                       pltpu.SemaphoreType.DMA,
                       pltpu.SemaphoreType.DMA,
                       pltpu.SemaphoreType.DMA,
                       pltpu.SemaphoreType.DMA],
    )
    def k(comb_hbm, ii2_hbm, cmax_hbm, init_hbm, out_hbm,
          ix0, ix1, ix2, ix3, bv0, bv1, py0, py1, cm_v, acc_sh,
          rs0, rs1, ss0, ss1):
        ixv = (ix0, ix1, ix2, ix3)
        bv = (bv0, bv1)
        pay = (py0, py1)
        rs = (rs0, rs1)
        ss = (ss0, ss1)
        c = lax.axis_index("c")
        sid = lax.axis_index("s")
        pltpu.sync_copy(init_hbm.at[c, pl.ds(sid * NPS, NPS)],
                        acc_sh.at[pl.ds(sid * NPS, NPS)])
        pltpu.sync_copy(cmax_hbm.at[c], cm_v)
        plsc.subcore_barrier()
        cms = [cm_v[pl.ds(g * 16, 16)] for g in range(Hh // 16)]

        def fire(kk, s2, s4):
            e0 = sid * ESC + kk * CKD
            r0 = sid * NCHD + kk
            pltpu.async_copy(ii2_hbm.at[pl.ds(r0, 1)], ixv[s4], rs[s2])
            pltpu.async_copy(comb_hbm.at[c, pl.ds(e0, CKD)], bv[s2], rs[s2])

        def drain_reads(s2, s4):
            pltpu.make_async_copy(ii2_hbm.at[pl.ds(0, 1)], ixv[s4],
                                  rs[s2]).wait()
            pltpu.make_async_copy(comb_hbm.at[c, pl.ds(0, CKD)], bv[s2],
                                  rs[s2]).wait()

        def compute(s2):
            def rowfn(r, cc):
                for rr in (2 * r, 2 * r + 1):
                    for g in range(Hh // 16):
                        ex = jnp.exp(bv[s2][rr, pl.ds(g * 16, 16)] - cms[g])
                        pay[s2][rr, pl.ds(Hh + g * 16, 16)] = ex
                        pay[s2][rr, pl.ds(g * 16, 16)] = \
                            ex * bv[s2][rr, pl.ds(Hh + g * 16, 16)]
                return cc

            lax.fori_loop(0, CKD // 2, rowfn, 0)

        def fire_scatter(s2, s4):
            pltpu.async_copy(pay[s2], acc_sh.at[ixv[s4].at[0]], ss[s2],
                             add=True)

        def drain_scatter(s2):
            pltpu.make_async_copy(pay[s2], acc_sh.at[ixv[0].at[0]],
                                  ss[s2]).wait()

        def phase(kk, s2, s4, first):
            drain_reads(s2, s4)
            if not first:
                drain_scatter(s2)
            compute(s2)
            fire_scatter(s2, s4)
            nxt = kk + 2

            @pl.when(nxt < NCHD)
            def _():
                fire(nxt, s2, (s4 + 2) % 4)

        fire(0, 0, 0)
        fire(1, 1, 1)
        phase(0, 0, 0, True)
        phase(1, 1, 1, True)
        phase(2, 0, 2, False)
        phase(3, 1, 3, False)

        def body(g, cc):
            for j in range(4):
                phase(4 * g + j, j % 2, j, False)
            return cc

        lax.fori_loop(1, NCHD // 4, body, 0)
        drain_scatter(0)
        drain_scatter(1)
        plsc.subcore_barrier()
        pltpu.sync_copy(acc_sh.at[pl.ds(sid * NPS, NPS)],
                        out_hbm.at[c, pl.ds(sid * NPS, NPS)])

    return k(comb, ii2, cmax2, init)


def _stage_e_body(acc_ref, alphaL_ref, sL_ref, cmax_ref, W_out_ref, b_out_ref,
                  o_ref):
    D = o_ref.shape[1]
    Hh = D // 2
    exL = jnp.exp(alphaL_ref[...] - cmax_ref[...])
    sL = sL_ref[...]
    num0 = acc_ref[0, :, 0:Hh] + exL[:, 0:Hh] * sL[:, 0:Hh]
    den0 = acc_ref[0, :, Hh:D] + exL[:, 0:Hh]
    num1 = acc_ref[1, :, 0:Hh] + exL[:, Hh:D] * sL[:, Hh:D]
    den1 = acc_ref[1, :, Hh:D] + exL[:, Hh:D]
    o0 = num0 / (den0 + 1e-16)
    o1 = num1 / (den1 + 1e-16)
    out = (o0 @ W_out_ref[0:Hh, :] + o1 @ W_out_ref[Hh:D, :]) + b_out_ref[...]
    o_ref[...] = jnp.maximum(out, 0.0)


def _stage_e(accsc, alphaL, sL, cmax_row, W_out, b_out):
    n, D = alphaL.shape
    grid = (n // BN,)
    full = lambda shape: pl.BlockSpec(shape, lambda i: (0, 0))
    row = lambda w: pl.BlockSpec((BN, w), lambda i: (i, 0))
    return pl.pallas_call(
        _stage_e_body,
        grid=grid,
        in_specs=[pl.BlockSpec((NC, BN, D), lambda i: (0, i, 0)),
                  row(D), row(D), full((1, D)), full((D, D)), full((1, D))],
        out_specs=row(D),
        out_shape=jax.ShapeDtypeStruct((n, D), jnp.float32),
    )(accsc, alphaL, sL, cmax_row, W_out, b_out)


def kernel(x, pos, edge_index, W_in, b_in, W_out, b_out, W_lin, W_src, W_dst,
           Wp1, bp1, Wp2, bp2, Wa1, ba1, Wa2, ba2):
    n, D = x.shape
    E = edge_index.shape[1]
    jj = edge_index[0].astype(jnp.int32)
    ii = edge_index[1].astype(jnp.int32)
    pos8 = jnp.pad(pos.astype(jnp.float32), ((0, 0), (0, 8 - pos.shape[1])))
    Wp1_p = jnp.pad(Wp1, ((0, 8 - Wp1.shape[0]), (0, D - Wp1.shape[1])))
    r1 = lambda v: v.reshape(1, -1)

    tbl, sL, alphaL, cmaxA = _stage_a(
        n, x, pos8, W_in, r1(b_in), W_src, W_dst, W_lin, Wp1_p, r1(bp1), Wp2,
        r1(bp2), Wa1, r1(ba1), Wa2, r1(ba2))
    cmax = jnp.max(cmaxA, axis=0)
    npad = ((n + NS * 8 - 1) // (NS * 8)) * (NS * 8)
    accsc = jnp.zeros((NC, npad, D), jnp.float32)

    unit = 32 * CK
    nu = E // unit
    u0 = max(1, (nu * 16) // 100)
    rest = nu - u0
    sl = [u0 * unit] + [(rest // 3 + (1 if t < rest % 3 else 0)) * unit
                        for t in range(3)]
    a0 = 0
    for Es in sl:
        iis = lax.dynamic_slice_in_dim(ii, a0, Es)
        jjs = lax.dynamic_slice_in_dim(jj, a0, Es)
        a0 += Es
        ijt = jnp.stack([iis.reshape(Es // CK, CK),
                         jjs.reshape(Es // CK, CK)], axis=1)
        hi, hj, qd = _gather_sc(tbl, ijt, Es)
        comb = _stage_c(
            hi, hj, qd, W_src, W_dst, W_lin, r1(bp1), Wp2, r1(bp2), Wa1,
            r1(ba1), Wa2, r1(ba2))
        accsc = _scatter_sc(comb, iis.reshape(Es // CKD, CKD),
                            cmax.reshape(NC, D // NC), accsc)
    return _stage_e(accsc, alphaL, sL, cmax.reshape(1, -1), W_out, r1(b_out))


# speedup vs baseline: 1.1643x; 1.1643x over previous
import functools

import jax
import jax.numpy as jnp
from jax import lax
from jax.experimental import pallas as pl
from jax.experimental.pallas import tpu as pltpu
from jax.experimental.pallas import tpu_sc as plsc

NC = 2
NS = 16
BN = 1000
KE = 1280
CK = 80
CKD = 40


def _stage_a_body(x_ref, pos_ref, W_in_ref, b_in_ref, W_src_ref, W_dst_ref,
                  W_lin_ref, Wp1_ref, bp1_ref, Wp2_ref, bp2_ref,
                  Wa1_ref, ba1_ref, Wa2_ref, ba2_ref,
                  t_ref, sL_ref, alphaL_ref, cmax_ref):
    pid = pl.program_id(0)
    D = x_ref.shape[1]
    h = jnp.maximum(x_ref[...] @ W_in_ref[...] + b_in_ref[...], 0.0)
    t_ref[:, 0:D] = h
    t_ref[:, D:2 * D] = pos_ref[...] @ Wp1_ref[...]
    d0 = jnp.maximum(jnp.maximum(bp1_ref[...], 0.0) @ Wp2_ref[...]
                     + bp2_ref[...], 0.0)
    sL_ref[...] = h @ W_lin_ref[...] + d0
    t = h @ W_dst_ref[...] - h @ W_src_ref[...] + d0
    a1 = jnp.maximum(t @ Wa1_ref[...] + ba1_ref[...], 0.0)
    aL = jnp.maximum(a1 @ Wa2_ref[...] + ba2_ref[...], 0.0)
    alphaL_ref[...] = aL

    @pl.when(pid == 0)
    def _():
        cmax_ref[...] = jnp.zeros_like(cmax_ref)

    cm = jnp.max(aL, axis=0, keepdims=True)
    cmax_ref[...] = jnp.maximum(cmax_ref[...], jnp.broadcast_to(cm, cmax_ref.shape))


def _stage_a(n, x, pos8, W_in, b_in, W_src, W_dst, W_lin, Wp1_p, bp1, Wp2, bp2,
             Wa1, ba1, Wa2, ba2):
    D = x.shape[1]
    H = bp1.shape[1]
    grid = (n // BN,)
    full = lambda shape: pl.BlockSpec(shape, lambda i: (0, 0))
    row = lambda w: pl.BlockSpec((BN, w), lambda i: (i, 0))
    return pl.pallas_call(
        _stage_a_body,
        grid=grid,
        in_specs=[row(D), pl.BlockSpec((BN, 8), lambda i: (i, 0)),
                  full((D, D)), full((1, D)), full((D, D)), full((D, D)),
                  full((D, D)), full((8, D)), full((1, H)), full((H, D)),
                  full((1, D)), full((D, H)), full((1, H)), full((H, D)),
                  full((1, D))],
        out_specs=[row(2 * D), row(D), row(D),
                   pl.BlockSpec((8, D), lambda i: (0, 0))],
        out_shape=[jax.ShapeDtypeStruct((n, 2 * D), jnp.float32),
                   jax.ShapeDtypeStruct((n, D), jnp.float32),
                   jax.ShapeDtypeStruct((n, D), jnp.float32),
                   jax.ShapeDtypeStruct((8, D), jnp.float32)],
    )(x, pos8, W_in, b_in, W_src, W_dst, W_lin, Wp1_p, bp1, Wp2, bp2,
      Wa1, ba1, Wa2, ba2)


def _gather_sc(tbl, ijt, E):
    n2, TW = tbl.shape
    D = TW // 2
    Hq = 64
    EW = E // (NC * NS)
    NCH = EW // CK

    @functools.partial(
        pl.kernel,
        out_type=[jax.ShapeDtypeStruct((E, D), jnp.float32),
                  jax.ShapeDtypeStruct((E, D), jnp.float32),
                  jax.ShapeDtypeStruct((E, Hq), jnp.float32)],
        mesh=plsc.VectorSubcoreMesh(core_axis_name="c", subcore_axis_name="s"),
        scratch_types=[pltpu.VMEM((2, CK), jnp.int32),
                       pltpu.VMEM((2, CK), jnp.int32),
                       pltpu.VMEM((2, CK), jnp.int32),
                       pltpu.VMEM((2, CK), jnp.int32),
                       pltpu.VMEM((CK, TW), jnp.float32),
                       pltpu.VMEM((CK, TW), jnp.float32),
                       pltpu.VMEM((CK, TW), jnp.float32),
                       pltpu.VMEM((CK, TW), jnp.float32),
                       pltpu.VMEM((CK, Hq), jnp.float32),
                       pltpu.VMEM((CK, Hq), jnp.float32),
                       pltpu.SemaphoreType.DMA,
                       pltpu.SemaphoreType.DMA,
                       pltpu.SemaphoreType.DMA,
                       pltpu.SemaphoreType.DMA,
                       pltpu.SemaphoreType.DMA,
                       pltpu.SemaphoreType.DMA],
    )
    def k(tbl_hbm, ijt_hbm, hi_hbm, hj_hbm, qd_hbm,
          ij0, ij1, ij2, ij3, bi0, bi1, bj0, bj1, qd0, qd1,
          gs0, gs1, ws0, ws1, js0, js1):
        ijv = (ij0, ij1, ij2, ij3)
        bi = (bi0, bi1)
        bj = (bj0, bj1)
        qdv = (qd0, qd1)
        gs = (gs0, gs1)
        ws = (ws0, ws1)
        js = (js0, js1)
        wid = lax.axis_index("s") * NC + lax.axis_index("c")
        base = wid * EW
        bch = wid * NCH

        def fire_ij(kk, sp, s4):
            pltpu.async_copy(ijt_hbm.at[bch + kk], ijv[s4], js[sp])

        def drain_ij(sp, s4):
            pltpu.make_async_copy(ijt_hbm.at[bch], ijv[s4], js[sp]).wait()

        def fire_gathers(kk, s2, s4):
            pltpu.async_copy(tbl_hbm.at[ijv[s4].at[0]], bi[s2], gs[s2])
            pltpu.async_copy(tbl_hbm.at[ijv[s4].at[1]], bj[s2], gs[s2])

        def drain_gathers(s2, s4):
            pltpu.make_async_copy(tbl_hbm.at[ijv[s4].at[0]], bi[s2],
                                  gs[s2]).wait()
            pltpu.make_async_copy(tbl_hbm.at[ijv[s4].at[1]], bj[s2],
                                  gs[s2]).wait()

        def qd_compute(s2):
            def rowfn(r, cc):
                for rr in (2 * r, 2 * r + 1):
                    for g in range(Hq // 16):
                        o = pl.ds(D + g * 16, 16)
                        qdv[s2][rr, pl.ds(g * 16, 16)] = \
                            bi[s2][rr, o] - bj[s2][rr, o]
                return cc

            lax.fori_loop(0, CK // 2, rowfn, 0)

        def fire_writes(kk, s2):
            e0 = base + kk * CK
            pltpu.async_copy(bi[s2].at[pl.ds(0, CK), pl.ds(0, D)],
                             hi_hbm.at[pl.ds(e0, CK)], ws[s2])
            pltpu.async_copy(bj[s2].at[pl.ds(0, CK), pl.ds(0, D)],
                             hj_hbm.at[pl.ds(e0, CK)], ws[s2])
            pltpu.async_copy(qdv[s2], qd_hbm.at[pl.ds(e0, CK)], ws[s2])

        def drain_writes(s2):
            pltpu.make_async_copy(bi[s2].at[pl.ds(0, CK), pl.ds(0, D)],
                                  hi_hbm.at[pl.ds(0, CK)], ws[s2]).wait()
            pltpu.make_async_copy(bj[s2].at[pl.ds(0, CK), pl.ds(0, D)],
                                  hj_hbm.at[pl.ds(0, CK)], ws[s2]).wait()
            pltpu.make_async_copy(qdv[s2], qd_hbm.at[pl.ds(0, CK)],
                                  ws[s2]).wait()

        def phase(kk, s2, s4, first):
            drain_gathers(s2, s4)
            nx1 = kk + 1

            @pl.when(nx1 < NCH)
            def _():
                drain_ij(1 - s2, (s4 + 1) % 4)
                if not first:
                    drain_writes(1 - s2)
                fire_gathers(nx1, 1 - s2, (s4 + 1) % 4)

            qd_compute(s2)
            fire_writes(kk, s2)
            nx2 = kk + 2

            @pl.when(nx2 < NCH)
            def _():
                fire_ij(nx2, s2, (s4 + 2) % 4)

        fire_ij(0, 0, 0)
        fire_ij(1, 1, 1)
        drain_ij(0, 0)
        fire_gathers(0, 0, 0)
        phase(0, 0, 0, True)
        phase(1, 1, 1, False)
        phase(2, 0, 2, False)
        phase(3, 1, 3, False)

        def body(g, cc):
            for j in range(4):
                phase(4 * g + j, j % 2, j, False)
            return cc

        lax.fori_loop(1, NCH // 4, body, 0)
        for kk in range(4 * (NCH // 4), NCH):
            phase(kk, kk % 2, kk % 4, False)
        drain_writes(0)
        drain_writes(1)

    return k(tbl, ijt)


def _stage_c_body(hi_ref, hj_ref, qd_ref, W_src_ref, W_dst_ref,
                  W_lin_ref, bp1_ref, Wp2_ref, bp2_ref, Wa1_ref, ba1_ref,
                  Wa2_ref, ba2_ref, comb_ref):
    hi = hi_ref[...]
    hj = hj_ref[...]
    g1 = qd_ref[...] + bp1_ref[...]
    delta = jnp.maximum(jnp.maximum(g1, 0.0) @ Wp2_ref[...] + bp2_ref[...], 0.0)
    t = hi @ W_dst_ref[...] - hj @ W_src_ref[...] + delta
    a1 = jnp.maximum(t @ Wa1_ref[...] + ba1_ref[...], 0.0)
    alpha = jnp.maximum(a1 @ Wa2_ref[...] + ba2_ref[...], 0.0)
    s = hj @ W_lin_ref[...] + delta
    Dh = alpha.shape[1] // 2
    comb_ref[0] = jnp.concatenate([alpha[:, 0:Dh], s[:, 0:Dh]], axis=1)
    comb_ref[1] = jnp.concatenate([alpha[:, Dh:], s[:, Dh:]], axis=1)


def _stage_c(hi, hj, qd, W_src, W_dst, W_lin, bp1, Wp2, bp2, Wa1, ba1,
             Wa2, ba2):
    E, D = hi.shape
    H = bp1.shape[1]
    grid = (E // KE,)
    full = lambda shape: pl.BlockSpec(shape, lambda i: (0, 0))
    row = lambda w: pl.BlockSpec((KE, w), lambda i: (i, 0))
    return pl.pallas_call(
        _stage_c_body,
        grid=grid,
        in_specs=[row(D), row(D), row(H),
                  full((D, D)), full((D, D)), full((D, D)), full((1, H)),
                  full((H, D)), full((1, D)), full((D, H)), full((1, H)),
                  full((H, D)), full((1, D))],
        out_specs=pl.BlockSpec((NC, KE, D), lambda i: (0, i, 0)),
        out_shape=jax.ShapeDtypeStruct((NC, E, D), jnp.float32),
    )(hi, hj, qd, W_src, W_dst, W_lin, bp1, Wp2, bp2, Wa1, ba1, Wa2, ba2)


def _scatter_sc(comb, ii2, cmax2, init):
    _, E, D = comb.shape
    Hh = D // 2
    n = init.shape[1]
    ESC = E // NS
    NCHD = ESC // CKD
    NPS = n // NS

    @functools.partial(
        pl.kernel,
        out_type=jax.ShapeDtypeStruct((NC, n, D), jnp.float32),
        mesh=plsc.VectorSubcoreMesh(core_axis_name="c", subcore_axis_name="s"),
        scratch_types=[pltpu.VMEM((1, CKD), jnp.int32),
                       pltpu.VMEM((1, CKD), jnp.int32),
                       pltpu.VMEM((1, CKD), jnp.int32),
                       pltpu.VMEM((1, CKD), jnp.int32),
                       pltpu.VMEM((CKD, D), jnp.float32),
                       pltpu.VMEM((CKD, D), jnp.float32),
                       pltpu.VMEM((CKD, D), jnp.float32),
                       pltpu.VMEM((CKD, D), jnp.float32),
                       pltpu.VMEM((Hh,), jnp.float32),
                       pltpu.VMEM_SHARED((n, D), jnp.float32),
                       pltpu.SemaphoreType.DMA,
                       pltpu.SemaphoreType.DMA,
                       pltpu.SemaphoreType.DMA,
                       pltpu.SemaphoreType.DMA],
    )
    def k(comb_hbm, ii2_hbm, cmax_hbm, init_hbm, out_hbm,
          ix0, ix1, ix2, ix3, bv0, bv1, py0, py1, cm_v, acc_sh,
          rs0, rs1, ss0, ss1):
        ixv = (ix0, ix1, ix2, ix3)
        bv = (bv0, bv1)
        pay = (py0, py1)
        rs = (rs0, rs1)
        ss = (ss0, ss1)
        c = lax.axis_index("c")
        sid = lax.axis_index("s")
        pltpu.sync_copy(init_hbm.at[c, pl.ds(sid * NPS, NPS)],
                        acc_sh.at[pl.ds(sid * NPS, NPS)])
        pltpu.sync_copy(cmax_hbm.at[c], cm_v)
        plsc.subcore_barrier()
        cms = [cm_v[pl.ds(g * 16, 16)] for g in range(Hh // 16)]

        def fire(kk, s2, s4):
            e0 = sid * ESC + kk * CKD
            r0 = sid * NCHD + kk
            pltpu.async_copy(ii2_hbm.at[pl.ds(r0, 1)], ixv[s4], rs[s2])
            pltpu.async_copy(comb_hbm.at[c, pl.ds(e0, CKD)], bv[s2], rs[s2])

        def drain_reads(s2, s4):
            pltpu.make_async_copy(ii2_hbm.at[pl.ds(0, 1)], ixv[s4],
                                  rs[s2]).wait()
            pltpu.make_async_copy(comb_hbm.at[c, pl.ds(0, CKD)], bv[s2],
                                  rs[s2]).wait()

        def compute(s2):
            def rowfn(r, cc):
                for rr in (2 * r, 2 * r + 1):
                    for g in range(Hh // 16):
                        ex = jnp.exp(bv[s2][rr, pl.ds(g * 16, 16)] - cms[g])
                        pay[s2][rr, pl.ds(Hh + g * 16, 16)] = ex
                        pay[s2][rr, pl.ds(g * 16, 16)] = \
                            ex * bv[s2][rr, pl.ds(Hh + g * 16, 16)]
                return cc

            lax.fori_loop(0, CKD // 2, rowfn, 0)

        def fire_scatter(s2, s4):
            pltpu.async_copy(pay[s2], acc_sh.at[ixv[s4].at[0]], ss[s2],
                             add=True)

        def drain_scatter(s2):
            pltpu.make_async_copy(pay[s2], acc_sh.at[ixv[0].at[0]],
                                  ss[s2]).wait()

        def phase(kk, s2, s4, first):
            drain_reads(s2, s4)
            if not first:
                drain_scatter(s2)
            compute(s2)
            fire_scatter(s2, s4)
            nxt = kk + 2

            @pl.when(nxt < NCHD)
            def _():
                fire(nxt, s2, (s4 + 2) % 4)

        fire(0, 0, 0)
        fire(1, 1, 1)
        phase(0, 0, 0, True)
        phase(1, 1, 1, True)
        phase(2, 0, 2, False)
        phase(3, 1, 3, False)

        def body(g, cc):
            for j in range(4):
                phase(4 * g + j, j % 2, j, False)
            return cc

        lax.fori_loop(1, NCHD // 4, body, 0)
        drain_scatter(0)
        drain_scatter(1)
        plsc.subcore_barrier()
        pltpu.sync_copy(acc_sh.at[pl.ds(sid * NPS, NPS)],
                        out_hbm.at[c, pl.ds(sid * NPS, NPS)])

    return k(comb, ii2, cmax2, init)


def _stage_e_body(acc_ref, alphaL_ref, sL_ref, cmax_ref, W_out_ref, b_out_ref,
                  o_ref):
    D = o_ref.shape[1]
    Hh = D // 2
    exL = jnp.exp(alphaL_ref[...] - cmax_ref[...])
    sL = sL_ref[...]
    num0 = acc_ref[0, :, 0:Hh] + exL[:, 0:Hh] * sL[:, 0:Hh]
    den0 = acc_ref[0, :, Hh:D] + exL[:, 0:Hh]
    num1 = acc_ref[1, :, 0:Hh] + exL[:, Hh:D] * sL[:, Hh:D]
    den1 = acc_ref[1, :, Hh:D] + exL[:, Hh:D]
    o0 = num0 / (den0 + 1e-16)
    o1 = num1 / (den1 + 1e-16)
    out = (o0 @ W_out_ref[0:Hh, :] + o1 @ W_out_ref[Hh:D, :]) + b_out_ref[...]
    o_ref[...] = jnp.maximum(out, 0.0)


def _stage_e(accsc, alphaL, sL, cmax_row, W_out, b_out):
    n, D = alphaL.shape
    grid = (n // BN,)
    full = lambda shape: pl.BlockSpec(shape, lambda i: (0, 0))
    row = lambda w: pl.BlockSpec((BN, w), lambda i: (i, 0))
    return pl.pallas_call(
        _stage_e_body,
        grid=grid,
        in_specs=[pl.BlockSpec((NC, BN, D), lambda i: (0, i, 0)),
                  row(D), row(D), full((1, D)), full((D, D)), full((1, D))],
        out_specs=row(D),
        out_shape=jax.ShapeDtypeStruct((n, D), jnp.float32),
    )(accsc, alphaL, sL, cmax_row, W_out, b_out)


def kernel(x, pos, edge_index, W_in, b_in, W_out, b_out, W_lin, W_src, W_dst,
           Wp1, bp1, Wp2, bp2, Wa1, ba1, Wa2, ba2):
    n, D = x.shape
    E = edge_index.shape[1]
    jj = edge_index[0].astype(jnp.int32)
    ii = edge_index[1].astype(jnp.int32)
    pos8 = jnp.pad(pos.astype(jnp.float32), ((0, 0), (0, 8 - pos.shape[1])))
    Wp1_p = jnp.pad(Wp1, ((0, 8 - Wp1.shape[0]), (0, D - Wp1.shape[1])))
    r1 = lambda v: v.reshape(1, -1)

    tbl, sL, alphaL, cmaxA = _stage_a(
        n, x, pos8, W_in, r1(b_in), W_src, W_dst, W_lin, Wp1_p, r1(bp1), Wp2,
        r1(bp2), Wa1, r1(ba1), Wa2, r1(ba2))
    cmax = jnp.max(cmaxA, axis=0)
    npad = ((n + NS * 8 - 1) // (NS * 8)) * (NS * 8)
    accsc = jnp.zeros((NC, npad, D), jnp.float32)

    unit = 32 * CK
    nu = E // unit
    u0 = max(1, (nu * 16) // 100)
    rest = nu - u0
    sl = [u0 * unit] + [(rest // 3 + (1 if t < rest % 3 else 0)) * unit
                        for t in range(3)]
    a0 = 0
    for Es in sl:
        iis = lax.dynamic_slice_in_dim(ii, a0, Es)
        jjs = lax.dynamic_slice_in_dim(jj, a0, Es)
        a0 += Es
        ijt = jnp.stack([iis.reshape(Es // CK, CK),
                         jjs.reshape(Es // CK, CK)], axis=1)
        hi, hj, qd = _gather_sc(tbl, ijt, Es)
        comb = _stage_c(
            hi, hj, qd, W_src, W_dst, W_lin, r1(bp1), Wp2, r1(bp2), Wa1,
            r1(ba1), Wa2, r1(ba2))
        accsc = _scatter_sc(comb, iis.reshape(Es // CKD, CKD),
                            cmax.reshape(NC, D // NC), accsc)
    return _stage_e(accsc, alphaL, sL, cmax.reshape(1, -1), W_out, r1(b_out))
